# baseline (speedup 1.0000x reference)
import functools
import math

import jax
import jax.numpy as jnp
from jax import lax
from jax.experimental import pallas as pl
from jax.experimental.pallas import tpu as pltpu
from jax.experimental.pallas import tpu_sc as plsc

_SMOOTH = 0.1
_SEQ_LEN = 4096
_ROWS = 1024
_LANES = 16
_GW = 128


def _make_sc_gather(n_tok, c):
    info = plsc.get_sparse_core_info()
    nc, ns = info.num_cores, info.num_subcores
    nw = nc * ns
    per_w = n_tok // nw
    rows_per_tok = c // _GW
    n_chunks = per_w // 128

    mesh = plsc.VectorSubcoreMesh(core_axis_name="c", subcore_axis_name="s")

    @functools.partial(
        pl.kernel,
        mesh=mesh,
        out_type=jax.ShapeDtypeStruct((n_tok, _GW), jnp.float32),
        scratch_types=[
            pltpu.VMEM((per_w,), jnp.int32),
            pltpu.VMEM((n_chunks, 128), jnp.int32),
            pltpu.VMEM((per_w, _GW), jnp.float32),
            pltpu.SemaphoreType.DMA,
        ],
    )
    def sc_gather(x_rows_hbm, tgt_hbm, out_hbm, tgt_v, ridx_v, rows_v, sem):
        wid = lax.axis_index("s") * nc + lax.axis_index("c")
        base = wid * per_w
        pltpu.sync_copy(tgt_hbm.at[pl.ds(base, per_w)], tgt_v)

        lane_ids = lax.iota(jnp.int32, _LANES)
        for j in range(per_w // _LANES):
            t = tgt_v[pl.ds(j * _LANES, _LANES)]
            tok = base + j * _LANES + lane_ids
            row = (lax.shift_right_logical(tok, 3) * (8 * rows_per_tok)
                   + lax.shift_right_logical(t, 7) * 8
                   + lax.bitwise_and(tok, 7))
            ridx_v[j * _LANES // 128, pl.ds((j * _LANES) % 128, _LANES)] = row

        copies = [
            pltpu.async_copy(
                x_rows_hbm.at[ridx_v.at[h]],
                rows_v.at[pl.ds(h * 128, 128)],
                sem,
            )
            for h in range(n_chunks)
        ]
        for cp in copies:
            cp.wait()

        pltpu.sync_copy(rows_v, out_hbm.at[pl.ds(base, per_w)])

    return sc_gather


def _dense_body(x_ref, o_ref):
    i = pl.program_id(0)

    @pl.when(i == 0)
    def _():
        o_ref[0, 0] = jnp.float32(0.0)

    o_ref[0, 0] += jnp.sum(jnp.log(x_ref[...]))


def _combine_body(g_ref, t_ref, s_ref, o_ref, *, fill, conf, t1):
    rows = g_ref[...]
    n, l = rows.shape
    lane = lax.bitwise_and(t_ref[...], l - 1)
    col = lax.broadcasted_iota(jnp.int32, (n, l), 1)
    vals = jnp.sum(jnp.where(col == lane, rows, 0.0), axis=1)
    s_gath = jnp.sum(jnp.log(vals))
    o_ref[0, 0] = t1 - fill * s_ref[0, 0] - (conf - fill) * s_gath


def kernel(x, target, device):
    b, s, c = x.shape
    n = b * s
    fill = _SMOOTH / _SEQ_LEN
    conf = 1.0 - _SMOOTH
    t1 = n * ((c - 1) * fill * math.log(fill) + conf * math.log(conf))

    x2 = x.reshape(n, c)
    x_rows = (x.reshape(n // 8, 8, c // _GW, _GW)
               .transpose(0, 2, 1, 3)
               .reshape(n * c // _GW, _GW))
    tgt_flat = target.reshape(n).astype(jnp.int32)

    g = _make_sc_gather(n, c)(x_rows, tgt_flat)

    nblk = n // _ROWS
    s_all = pl.pallas_call(
        _dense_body,
        grid=(nblk,),
        in_specs=[pl.BlockSpec((_ROWS, c), lambda i: (i, 0))],
        out_specs=pl.BlockSpec((1, 1), lambda i: (0, 0),
                               memory_space=pltpu.SMEM),
        out_shape=jax.ShapeDtypeStruct((1, 1), jnp.float32),
    )(x2)

    out = pl.pallas_call(
        functools.partial(_combine_body, fill=fill, conf=conf, t1=t1),
        in_specs=[
            pl.BlockSpec((n, _GW), lambda: (0, 0)),
            pl.BlockSpec((n, 1), lambda: (0, 0)),
            pl.BlockSpec(memory_space=pltpu.SMEM),
        ],
        out_specs=pl.BlockSpec(memory_space=pltpu.SMEM),
        out_shape=jax.ShapeDtypeStruct((1, 1), jnp.float32),
    )(g, tgt_flat.reshape(n, 1), s_all)
    return out[0, 0]


# speedup vs baseline: 1.5609x; 1.5609x over previous
import functools
import math

import jax
import jax.numpy as jnp
from jax.experimental import pallas as pl
from jax.experimental.pallas import tpu as pltpu

_SMOOTH = 0.1
_SEQ_LEN = 4096
_ROWS = 1024


def _body(x_ref, t_ref, o_ref, *, fill, conf, t1):
    i = pl.program_id(0)
    r, c = x_ref.shape
    tgt = t_ref[0, 0, :].reshape(r, 1)
    col = jax.lax.broadcasted_iota(jnp.int32, (r, c), 1)
    w = jnp.where(col == tgt, jnp.float32(conf), jnp.float32(fill))
    part = jnp.sum(w * jnp.log(x_ref[...]))

    @pl.when(i == 0)
    def _():
        o_ref[0, 0] = jnp.float32(t1)

    o_ref[0, 0] = o_ref[0, 0] - part


def kernel(x, target, device):
    b, s, c = x.shape
    n = b * s
    fill = _SMOOTH / _SEQ_LEN
    conf = 1.0 - _SMOOTH
    t1 = n * ((c - 1) * fill * math.log(fill) + conf * math.log(conf))

    x2 = x.reshape(n, c)
    nblk = n // _ROWS
    t3 = target.reshape(nblk, 1, _ROWS).astype(jnp.int32)

    body = functools.partial(_body, fill=fill, conf=conf, t1=t1)
    out = pl.pallas_call(
        body,
        grid=(nblk,),
        in_specs=[
            pl.BlockSpec((_ROWS, c), lambda i: (i, 0)),
            pl.BlockSpec((1, 1, _ROWS), lambda i: (i, 0, 0)),
        ],
        out_specs=pl.BlockSpec((1, 1), lambda i: (0, 0),
                               memory_space=pltpu.SMEM),
        out_shape=jax.ShapeDtypeStruct((1, 1), jnp.float32),
    )(x2, t3)
    return out[0, 0]
